# baseline (speedup 1.0000x reference)
import functools

import jax
import jax.numpy as jnp
from jax import lax
from jax.experimental import pallas as pl
from jax.experimental.pallas import tpu as pltpu
from jax.experimental.pallas import tpu_sc as plsc

MASK = 17
STRIDE = 4
B, C, H, W = 8, 3, 257, 257
NX = (H - MASK) // STRIDE + 1
NY = (W - MASK) // STRIDE + 1
NW = 32
NITEMS = C * NX
T_STEPS = (NITEMS + NW - 1) // NW
XSEG = MASK * W + 7
XSTEP = XSEG + 8
IY0S = (0, 16, 32, 45)
QP = 68
RSTEP = 4 * QP


def _sc_body(x_hbm, out_hbm, xs, xr, ob, sem0, sem1):
    wid = lax.axis_index("s") * 2 + lax.axis_index("c")
    lanes = lax.iota(jnp.int32, 16)
    idxw0 = (lanes % 4) * QP + lanes // 4
    c1 = W - 16
    idxw1 = ((lanes + c1) % 4) * QP + (lanes + c1) // 4 - (c1 // 4)

    def item_step(t, carry):
        it = wid + NW * t

        @pl.when(it < NITEMS)
        def _():
            c = it // NX
            ix = it - c * NX
            bd = []
            for b in range(B):
                src0 = ((b * C + c) * H + STRIDE * ix) * W
                srca = (src0 // 8) * 8
                pltpu.sync_copy(x_hbm.at[pl.ds(srca, XSEG)],
                                xs.at[pl.ds(b * XSTEP, XSEG)])
                bd.append(b * XSTEP + (src0 - srca))

            @plsc.parallel_loop(0, MASK, unroll=4)
            def _swz(u):
                for b in range(B):
                    rbase = bd[b] + u * W
                    wbase = (b * MASK + u) * RSTEP
                    for k in range(16):
                        vals = xs[pl.ds(rbase + 16 * k, 16)]
                        plsc.store_scatter(xr, [idxw0 + (wbase + 4 * k)], vals)
                    vals = xs[pl.ds(rbase + c1, 16)]
                    plsc.store_scatter(xr, [idxw1 + (wbase + c1 // 4)], vals)

            def u_step(u, carry2):
                par = u % 2
                dst = out_hbm.at[c, ix, u]

                @pl.when(u >= 2)
                def _():
                    @pl.when(par == 0)
                    def _():
                        pltpu.make_async_copy(ob.at[0], dst, sem0).wait()

                    @pl.when(par == 1)
                    def _():
                        pltpu.make_async_copy(ob.at[1], dst, sem1).wait()

                ur = u * RSTEP

                @plsc.parallel_loop(0, MASK, unroll=4)
                def _build(v):
                    vq = (v % 4) * QP + v // 4
                    for b in range(B):
                        vb = b * MASK * RSTEP + vq
                        for iy0 in IY0S:
                            ob[par, v, b, pl.ds(iy0, 16)] = (
                                xr[pl.ds(ur + vb + iy0, 16)])

                @pl.when(par == 0)
                def _():
                    pltpu.async_copy(ob.at[0], dst, sem0)

                @pl.when(par == 1)
                def _():
                    pltpu.async_copy(ob.at[1], dst, sem1)

                return carry2

            lax.fori_loop(0, MASK, u_step, 0)
            pltpu.make_async_copy(ob.at[0], out_hbm.at[c, ix, 16], sem0).wait()
            pltpu.make_async_copy(ob.at[1], out_hbm.at[c, ix, 15], sem1).wait()

        return carry

    lax.fori_loop(0, T_STEPS, item_step, 0)


def kernel(x, bins):
    del bins
    x_flat = x.reshape(B * C * H * W)
    mesh = plsc.VectorSubcoreMesh(core_axis_name="c", subcore_axis_name="s")
    run = functools.partial(
        pl.kernel,
        mesh=mesh,
        compiler_params=pltpu.CompilerParams(needs_layout_passes=False),
        out_type=jax.ShapeDtypeStruct((C, NX, MASK, MASK, B, NY), jnp.float32),
        scratch_types=[
            pltpu.VMEM((B * XSTEP,), jnp.float32),
            pltpu.VMEM((B * MASK * RSTEP,), jnp.float32),
            pltpu.VMEM((2, MASK, B, NY), jnp.float32),
            pltpu.SemaphoreType.DMA,
            pltpu.SemaphoreType.DMA,
        ],
    )(_sc_body)
    out_phys = run(x_flat)
    return jnp.transpose(out_phys, (4, 0, 1, 5, 2, 3))


# speedup vs baseline: 1.0141x; 1.0141x over previous
import functools

import jax
import jax.numpy as jnp
from jax import lax
from jax.experimental import pallas as pl
from jax.experimental.pallas import tpu as pltpu
from jax.experimental.pallas import tpu_sc as plsc

MASK = 17
STRIDE = 4
B, C, H, W = 8, 3, 257, 257
NX = (H - MASK) // STRIDE + 1
NY = (W - MASK) // STRIDE + 1
NW = 32
NITEMS = C * NX
T_STEPS = (NITEMS + NW - 1) // NW
XSEG = MASK * W + 7
XSTEP = XSEG + 8
IY0S = (0, 16, 32, 45)
QP = 68
RSTEP = 4 * QP


def _sc_body(x_hbm, out_hbm, xs, xr, ob, sem0, sem1):
    wid = lax.axis_index("s") * 2 + lax.axis_index("c")
    lanes = lax.iota(jnp.int32, 16)
    idxw0 = (lanes % 4) * QP + lanes // 4
    c1 = W - 16
    idxw1 = ((lanes + c1) % 4) * QP + (lanes + c1) // 4 - (c1 // 4)

    def item_step(t, carry):
        it = wid + NW * t

        @pl.when(it < NITEMS)
        def _():
            c = it // NX
            ix = it - c * NX
            bd = []
            for b in range(B):
                src0 = ((b * C + c) * H + STRIDE * ix) * W
                srca = (src0 // 8) * 8
                pltpu.sync_copy(x_hbm.at[pl.ds(srca, XSEG)],
                                xs.at[pl.ds(b * XSTEP, XSEG)])
                bd.append(b * XSTEP + (src0 - srca))

            @plsc.parallel_loop(0, MASK, unroll=2)
            def _swz(u):
                for b in range(B):
                    rbase = bd[b] + u * W
                    wbase = (b * MASK + u) * RSTEP
                    for k in range(16):
                        vals = xs[pl.ds(rbase + 16 * k, 16)]
                        plsc.store_scatter(xr, [idxw0 + (wbase + 4 * k)], vals)
                    vals = xs[pl.ds(rbase + c1, 16)]
                    plsc.store_scatter(xr, [idxw1 + (wbase + c1 // 4)], vals)

            def u_step(u, carry2):
                par = u % 2
                dst = out_hbm.at[c, ix, u]

                @pl.when(u >= 2)
                def _():
                    @pl.when(par == 0)
                    def _():
                        pltpu.make_async_copy(ob.at[0], dst, sem0).wait()

                    @pl.when(par == 1)
                    def _():
                        pltpu.make_async_copy(ob.at[1], dst, sem1).wait()

                ur = u * RSTEP

                @plsc.parallel_loop(0, MASK, unroll=2)
                def _build(v):
                    vq = (v % 4) * QP + v // 4
                    for b in range(B):
                        vb = b * MASK * RSTEP + vq
                        for iy0 in IY0S:
                            ob[par, v, b, pl.ds(iy0, 16)] = (
                                xr[pl.ds(ur + vb + iy0, 16)])

                @pl.when(par == 0)
                def _():
                    pltpu.async_copy(ob.at[0], dst, sem0)

                @pl.when(par == 1)
                def _():
                    pltpu.async_copy(ob.at[1], dst, sem1)

                return carry2

            lax.fori_loop(0, MASK, u_step, 0)
            pltpu.make_async_copy(ob.at[0], out_hbm.at[c, ix, 16], sem0).wait()
            pltpu.make_async_copy(ob.at[1], out_hbm.at[c, ix, 15], sem1).wait()

        return carry

    lax.fori_loop(0, T_STEPS, item_step, 0)


def kernel(x, bins):
    del bins
    x_flat = x.reshape(B * C * H * W)
    mesh = plsc.VectorSubcoreMesh(core_axis_name="c", subcore_axis_name="s")
    run = functools.partial(
        pl.kernel,
        mesh=mesh,
        compiler_params=pltpu.CompilerParams(needs_layout_passes=False),
        out_type=jax.ShapeDtypeStruct((C, NX, MASK, MASK, B, NY), jnp.float32),
        scratch_types=[
            pltpu.VMEM((B * XSTEP,), jnp.float32),
            pltpu.VMEM((B * MASK * RSTEP,), jnp.float32),
            pltpu.VMEM((2, MASK, B, NY), jnp.float32),
            pltpu.SemaphoreType.DMA,
            pltpu.SemaphoreType.DMA,
        ],
    )(_sc_body)
    out_phys = run(x_flat)
    return jnp.transpose(out_phys, (4, 0, 1, 5, 2, 3))
